# Initial kernel scaffold; baseline (speedup 1.0000x reference)
import jax
import jax.numpy as jnp
from jax.experimental import pallas as pl


def kernel(x0, x1, x2, x3, edge_index0, edge_index1, edge_index2, edge_index3, log_nA, delta, Wl0, Wr0, bc0, Wl1, Wr1, bc1, Wres0, bres0, Wres1, bres1, wmix, bmix, Wmlp, bmlp):
    raise NotImplementedError("write your pallas kernel here")


# speedup vs baseline: 2.9691x; 2.9691x over previous
import functools

import jax
import jax.numpy as jnp
from jax import lax
from jax.experimental import pallas as pl
from jax.experimental.pallas import tpu as pltpu
from jax.experimental.pallas import tpu_sc as plsc

V = 10000
E = 320000
D = 128
N = 16
T = 4
OUT = 2 * D + 2 * N

NC = 2
NS = 16
NW = NC * NS
EPW = E // NW
KB = 80
NBLK = EPW // KB
RPS = 624
ZCH = 8
NZ = RPS // ZCH
TAIL = V - RPS * NS
OCH = 80
NOC = RPS // OCH


def _zero_acc(s, zb, agg_sh):
    r0 = s * RPS

    def zchunk(i, carry):
        pltpu.sync_copy(zb, agg_sh.at[pl.ds(r0 + i * ZCH, ZCH)])
        return carry

    lax.fori_loop(0, NZ, zchunk, 0)

    @pl.when(s == NS - 1)
    def _zero_tail():
        for tt in range(TAIL // ZCH):
            pltpu.sync_copy(zb, agg_sh.at[pl.ds(RPS * NS + tt * ZCH, ZCH)])


def _copy_out(s, c, j, rows, agg_sh, out):
    r0 = s * RPS

    def ochunk(i, carry):
        rr = r0 + i * OCH
        pltpu.sync_copy(agg_sh.at[pl.ds(rr, OCH)], rows)
        pltpu.sync_copy(rows, out.at[c, j].at[pl.ds(rr, OCH)])
        return carry

    lax.fori_loop(0, NOC, ochunk, 0)
    rem = RPS - NOC * OCH
    rr = r0 + NOC * OCH
    pltpu.sync_copy(agg_sh.at[pl.ds(rr, rem)], rows.at[:rem])
    pltpu.sync_copy(rows.at[:rem], out.at[c, j].at[pl.ds(rr, rem)])

    @pl.when(s == NS - 1)
    def _copy_tail():
        rr2 = RPS * NS
        pltpu.sync_copy(agg_sh.at[pl.ds(rr2, TAIL)], rows.at[:TAIL])
        pltpu.sync_copy(rows.at[:TAIL], out.at[c, j].at[pl.ds(rr2, TAIL)])


@functools.lru_cache(maxsize=None)
def _make_sc_segsum(with_deg):
    mesh = plsc.VectorSubcoreMesh(core_axis_name="c", subcore_axis_name="s")
    out_type = [jax.ShapeDtypeStruct((NC, T, V, D), jnp.float32)]
    if with_deg:
        out_type.append(jax.ShapeDtypeStruct((NC, T, V, D), jnp.float32))
    scratch = [
        pltpu.VMEM((KB,), jnp.int32),
        pltpu.VMEM((KB,), jnp.int32),
        pltpu.VMEM((KB, D), jnp.float32),
        pltpu.VMEM((KB, D), jnp.float32),
        pltpu.VMEM((ZCH, D), jnp.float32),
        pltpu.VMEM_SHARED((V, D), jnp.float32),
        pltpu.SemaphoreType.DMA,
    ]

    @functools.partial(pl.kernel, mesh=mesh, out_type=out_type,
                       scratch_types=scratch)
    def seg_kernel(x0, x1, x2, x3, s0, s1, s2, s3, d0, d1, d2, d3,
                   zrows, ones128, *rest):
        if with_deg:
            aggp, degp = rest[0], rest[1]
            rest = rest[2:]
        else:
            aggp = rest[0]
            degp = None
            rest = rest[1:]
        (sidx, didx, rows, onesb, zb, agg_sh, sem) = rest
        c = lax.axis_index("c")
        s = lax.axis_index("s")
        w = c * NS + s
        pltpu.sync_copy(zrows, zb)
        if with_deg:
            pltpu.sync_copy(ones128, onesb)
        xs = (x0, x1, x2, x3)
        ss = (s0, s1, s2, s3)
        ds = (d0, d1, d2, d3)
        ebase = w * EPW
        for j in range(T):
            _zero_acc(s, zb, agg_sh)
            plsc.subcore_barrier()

            def blk(b, carry):
                off = ebase + b * KB
                pltpu.sync_copy(ss[j].at[pl.ds(off, KB)], sidx)
                pltpu.sync_copy(ds[j].at[pl.ds(off, KB)], didx)
                pltpu.async_copy(xs[j].at[sidx], rows, sem).wait()
                pltpu.sync_copy(rows, agg_sh.at[didx], add=True)
                return carry

            lax.fori_loop(0, NBLK, blk, 0)
            plsc.subcore_barrier()
            _copy_out(s, c, j, rows, agg_sh, aggp)
            if with_deg:
                plsc.subcore_barrier()
                _zero_acc(s, zb, agg_sh)
                plsc.subcore_barrier()

                def dblk(b, carry):
                    off = ebase + b * KB
                    pltpu.sync_copy(ds[j].at[pl.ds(off, KB)], didx)
                    pltpu.sync_copy(onesb, agg_sh.at[didx], add=True)
                    return carry

                lax.fori_loop(0, NBLK, dblk, 0)
                plsc.subcore_barrier()
                _copy_out(s, c, j, rows, agg_sh, degp)
            plsc.subcore_barrier()

    return seg_kernel


def _softplus(x):
    return jnp.log1p(jnp.exp(-jnp.abs(x))) + jnp.maximum(x, 0.0)


def _dense_body(layer, xs, aggblk, degblk, Wl, Wr, bc, Wres, bres,
                delta, AT, wmix, bmix, Wmlp, bmlp):
    xsr = [x @ Wres + bres for x in xs]
    nx, dts, Bs, Cs = [], [], [], []
    for j in range(T):
        deg = degblk[0, j, :, 0:1] + degblk[1, j, :, 0:1]
        rdeg = 1.0 / jnp.clip(deg, 1.0)
        agg = (aggblk[0, j] + aggblk[1, j]) * rdeg
        o = agg @ Wl + xs[j] @ Wr + bc
        nx.append(o[:, :D])
        dts.append(o[:, D:2 * D])
        Bs.append(o[:, 2 * D:2 * D + N])
        Cs.append(o[:, 2 * D + N:])
    if layer == 0:
        xs_ = []
        for t in range(T):
            y = nx[t] * wmix[1:2, :] + bmix
            if t - 1 >= 0:
                y = y + nx[t - 1] * wmix[0:1, :]
            if t + 1 < T:
                y = y + nx[t + 1] * wmix[2:3, :]
            xs_.append(y)
    else:
        xs_ = nx
    b = xs[0].shape[0]
    state = [jnp.zeros((b, D), jnp.float32) for _ in range(N)]
    outl = []
    for j in range(T):
        dt = _softplus(dts[j] + delta)
        dtx = dt * xs_[j]
        need_y = (layer == 0) or (j == T - 1)
        y = jnp.zeros((b, D), jnp.float32)
        for n in range(N):
            Az = jnp.exp(dt * AT[n:n + 1, :])
            st = Az * state[n] + dtx * Bs[j][:, n:n + 1]
            state[n] = st
            if need_y:
                y = y + st * Cs[j][:, n:n + 1]
        if need_y:
            y = jnp.maximum(y, 0.0) + xsr[j]
            m = jnp.mean(y, axis=-1, keepdims=True)
            var = jnp.mean((y - m) ** 2, axis=-1, keepdims=True)
            outl.append((y - m) * lax.rsqrt(var + 1e-5))
        else:
            outl.append(None)
    if layer == 0:
        return outl
    return outl[-1] @ Wmlp + bmlp


BV = 400
GRID = V // BV


def _make_tc_layer(layer):
    def body(x0r, x1r, x2r, x3r, aggr, degr, Wlr, Wrr, bcr, Wresr, bresr,
             deltar, lognATr, wmixr, bmixr, Wmlpr, bmlpr, *outs):
        xs = [x0r[...], x1r[...], x2r[...], x3r[...]]
        AT = -jnp.exp(lognATr[...])
        res = _dense_body(layer, xs, aggr[...], degr[...], Wlr[...], Wrr[...],
                          bcr[...], Wresr[...], bresr[...], deltar[...], AT,
                          wmixr[...], bmixr[...], Wmlpr[...], bmlpr[...])
        if layer == 0:
            for o_ref, o in zip(outs, res):
                o_ref[...] = o
        else:
            outs[0][...] = res

    bspec = pl.BlockSpec((BV, D), lambda i: (i, 0))
    full = lambda shape: pl.BlockSpec(shape, lambda i: tuple(0 for _ in shape))
    in_specs = [
        bspec, bspec, bspec, bspec,
        pl.BlockSpec((NC, T, BV, D), lambda i: (0, 0, i, 0)),
        pl.BlockSpec((NC, T, BV, D), lambda i: (0, 0, i, 0)),
        full((D, OUT)), full((D, OUT)), full((1, OUT)),
        full((D, D)), full((1, D)),
        full((1, D)), full((N, D)), full((3, D)), full((1, D)),
        full((D, D)), full((1, D)),
    ]
    if layer == 0:
        out_shape = [jax.ShapeDtypeStruct((V, D), jnp.float32)] * T
        out_specs = [bspec] * T
    else:
        out_shape = jax.ShapeDtypeStruct((V, D), jnp.float32)
        out_specs = bspec
    return pl.pallas_call(
        body, grid=(GRID,), in_specs=in_specs, out_specs=out_specs,
        out_shape=out_shape,
        compiler_params=pltpu.CompilerParams(
            dimension_semantics=("arbitrary",)),
    )


_tc_layer0 = _make_tc_layer(0)
_tc_layer1 = _make_tc_layer(1)


def kernel(x0, x1, x2, x3, edge_index0, edge_index1, edge_index2, edge_index3,
           log_nA, delta, Wl0, Wr0, bc0, Wl1, Wr1, bc1, Wres0, bres0,
           Wres1, bres1, wmix, bmix, Wmlp, bmlp):
    eis = (edge_index0, edge_index1, edge_index2, edge_index3)
    srcs = [ei[0] for ei in eis]
    dsts = [ei[1] for ei in eis]
    zrows = jnp.zeros((ZCH, D), jnp.float32)
    ones128 = jnp.ones((KB, D), jnp.float32)

    aggp0, degp = _make_sc_segsum(True)(x0, x1, x2, x3, *srcs, *dsts,
                                        zrows, ones128)

    r2 = lambda b: b.reshape(1, -1)
    ys = _tc_layer0(x0, x1, x2, x3, aggp0, degp,
                    Wl0, Wr0, r2(bc0), Wres0, r2(bres0),
                    r2(delta[0]), log_nA[0].T, wmix, r2(bmix), Wmlp, r2(bmlp))
    y0, y1, y2, y3 = ys

    aggp1 = _make_sc_segsum(False)(y0, y1, y2, y3, *srcs, *dsts,
                                   zrows, ones128)
    if isinstance(aggp1, (list, tuple)):
        aggp1 = aggp1[0]

    out = _tc_layer1(y0, y1, y2, y3, aggp1, degp,
                     Wl1, Wr1, r2(bc1), Wres1, r2(bres1),
                     r2(delta[1]), log_nA[1].T, wmix, r2(bmix), Wmlp, r2(bmlp))
    return out


# speedup vs baseline: 4.2130x; 1.4189x over previous
import functools

import jax
import jax.numpy as jnp
from jax import lax
from jax.experimental import pallas as pl
from jax.experimental.pallas import tpu as pltpu
from jax.experimental.pallas import tpu_sc as plsc

V = 10000
E = 320000
D = 128
N = 16
T = 4
OUT = 2 * D + 2 * N

NC = 2
NS = 16
NW = NC * NS
EPW = E // NW
KB = 80
NBLK = EPW // KB
RPS = 624
ZCH = 8
NZ = RPS // ZCH
TAIL = V - RPS * NS
OCH = 80
NOC = RPS // OCH


def _zero_acc(s, zb, agg_sh):
    r0 = s * RPS

    def zchunk(i, carry):
        pltpu.sync_copy(zb, agg_sh.at[pl.ds(r0 + i * ZCH, ZCH)])
        return carry

    lax.fori_loop(0, NZ, zchunk, 0)

    @pl.when(s == NS - 1)
    def _zero_tail():
        for tt in range(TAIL // ZCH):
            pltpu.sync_copy(zb, agg_sh.at[pl.ds(RPS * NS + tt * ZCH, ZCH)])


def _copy_out(s, c, j, rows, agg_sh, out):
    r0 = s * RPS

    def ochunk(i, carry):
        rr = r0 + i * OCH
        pltpu.sync_copy(agg_sh.at[pl.ds(rr, OCH)], rows)
        pltpu.sync_copy(rows, out.at[c, j].at[pl.ds(rr, OCH)])
        return carry

    lax.fori_loop(0, NOC, ochunk, 0)
    rem = RPS - NOC * OCH
    rr = r0 + NOC * OCH
    pltpu.sync_copy(agg_sh.at[pl.ds(rr, rem)], rows.at[:rem])
    pltpu.sync_copy(rows.at[:rem], out.at[c, j].at[pl.ds(rr, rem)])

    @pl.when(s == NS - 1)
    def _copy_tail():
        rr2 = RPS * NS
        pltpu.sync_copy(agg_sh.at[pl.ds(rr2, TAIL)], rows.at[:TAIL])
        pltpu.sync_copy(rows.at[:TAIL], out.at[c, j].at[pl.ds(rr2, TAIL)])


@functools.lru_cache(maxsize=None)
def _make_sc_segsum(with_deg):
    mesh = plsc.VectorSubcoreMesh(core_axis_name="c", subcore_axis_name="s")
    out_type = [jax.ShapeDtypeStruct((NC, T, V, D), jnp.float32)]
    if with_deg:
        out_type.append(jax.ShapeDtypeStruct((NC, T, V, D), jnp.float32))
    scratch = [
        pltpu.VMEM((KB,), jnp.int32),
        pltpu.VMEM((KB,), jnp.int32),
        pltpu.VMEM((KB,), jnp.int32),
        pltpu.VMEM((KB,), jnp.int32),
        pltpu.VMEM((KB, D), jnp.float32),
        pltpu.VMEM((KB, D), jnp.float32),
        pltpu.VMEM((ZCH, D), jnp.float32),
        pltpu.VMEM_SHARED((V, D), jnp.float32),
        pltpu.SemaphoreType.DMA,
        pltpu.SemaphoreType.DMA,
    ]

    @functools.partial(pl.kernel, mesh=mesh, out_type=out_type,
                       scratch_types=scratch)
    def seg_kernel(x0, x1, x2, x3, s0, s1, s2, s3, d0, d1, d2, d3,
                   zrows, ones128, *rest):
        if with_deg:
            aggp, degp = rest[0], rest[1]
            rest = rest[2:]
        else:
            aggp = rest[0]
            degp = None
            rest = rest[1:]
        (sidx0, didx0, sidx1, didx1, rows0, rows1, zb,
         agg_sh, sem0, sem1) = rest
        c = lax.axis_index("c")
        s = lax.axis_index("s")
        w = c * NS + s
        pltpu.sync_copy(zrows, zb)
        xs = (x0, x1, x2, x3)
        ss = (s0, s1, s2, s3)
        ds = (d0, d1, d2, d3)
        ebase = w * EPW
        for j in range(T):
            _zero_acc(s, zb, agg_sh)
            plsc.subcore_barrier()

            pltpu.sync_copy(ss[j].at[pl.ds(ebase, KB)], sidx0)
            pltpu.sync_copy(ds[j].at[pl.ds(ebase, KB)], didx0)
            pltpu.async_copy(xs[j].at[sidx0], rows0, sem0)

            def pair(i, carry):
                o1 = ebase + (2 * i + 1) * KB
                pltpu.sync_copy(ss[j].at[pl.ds(o1, KB)], sidx1)
                pltpu.sync_copy(ds[j].at[pl.ds(o1, KB)], didx1)
                pltpu.async_copy(xs[j].at[sidx1], rows1, sem1)
                pltpu.make_async_copy(xs[j].at[pl.ds(0, KB)], rows0,
                                      sem0).wait()
                pltpu.sync_copy(rows0, agg_sh.at[didx0], add=True)
                o2 = o1 + KB
                pltpu.sync_copy(ss[j].at[pl.ds(o2, KB)], sidx0)
                pltpu.sync_copy(ds[j].at[pl.ds(o2, KB)], didx0)
                pltpu.async_copy(xs[j].at[sidx0], rows0, sem0)
                pltpu.make_async_copy(xs[j].at[pl.ds(0, KB)], rows1,
                                      sem1).wait()
                pltpu.sync_copy(rows1, agg_sh.at[didx1], add=True)
                return carry

            lax.fori_loop(0, NBLK // 2, pair, 0)
            pltpu.make_async_copy(xs[j].at[pl.ds(0, KB)], rows0, sem0).wait()
            pltpu.sync_copy(rows0, agg_sh.at[didx0], add=True)
            plsc.subcore_barrier()
            _copy_out(s, c, j, rows0, agg_sh, aggp)
            if with_deg:
                plsc.subcore_barrier()
                _zero_acc(s, zb, agg_sh)
                plsc.subcore_barrier()
                pltpu.sync_copy(ones128, rows0)

                def dblk(b, carry):
                    off = ebase + b * KB
                    pltpu.sync_copy(ds[j].at[pl.ds(off, KB)], didx0)
                    pltpu.sync_copy(rows0, agg_sh.at[didx0], add=True)
                    return carry

                lax.fori_loop(0, NBLK, dblk, 0)
                plsc.subcore_barrier()
                _copy_out(s, c, j, rows1, agg_sh, degp)
            plsc.subcore_barrier()

    return seg_kernel


def _softplus(x):
    return jnp.log1p(jnp.exp(-jnp.abs(x))) + jnp.maximum(x, 0.0)


def _dense_body(layer, xs, aggblk, degblk, Wl, Wr, bc, Wres, bres,
                delta, AT, wmix, bmix, Wmlp, bmlp):
    xsr = [x @ Wres + bres for x in xs]
    nx, dts, Bs, Cs = [], [], [], []
    for j in range(T):
        deg = degblk[0, j, :, 0:1] + degblk[1, j, :, 0:1]
        rdeg = 1.0 / jnp.clip(deg, 1.0)
        agg = (aggblk[0, j] + aggblk[1, j]) * rdeg
        o = agg @ Wl + xs[j] @ Wr + bc
        nx.append(o[:, :D])
        dts.append(o[:, D:2 * D])
        Bs.append(o[:, 2 * D:2 * D + N])
        Cs.append(o[:, 2 * D + N:])
    if layer == 0:
        xs_ = []
        for t in range(T):
            y = nx[t] * wmix[1:2, :] + bmix
            if t - 1 >= 0:
                y = y + nx[t - 1] * wmix[0:1, :]
            if t + 1 < T:
                y = y + nx[t + 1] * wmix[2:3, :]
            xs_.append(y)
    else:
        xs_ = nx
    b = xs[0].shape[0]
    state = [jnp.zeros((b, D), jnp.float32) for _ in range(N)]
    outl = []
    for j in range(T):
        dt = _softplus(dts[j] + delta)
        dtx = dt * xs_[j]
        need_y = (layer == 0) or (j == T - 1)
        y = jnp.zeros((b, D), jnp.float32)
        for n in range(N):
            Az = jnp.exp(dt * AT[n:n + 1, :])
            st = Az * state[n] + dtx * Bs[j][:, n:n + 1]
            state[n] = st
            if need_y:
                y = y + st * Cs[j][:, n:n + 1]
        if need_y:
            y = jnp.maximum(y, 0.0) + xsr[j]
            m = jnp.mean(y, axis=-1, keepdims=True)
            var = jnp.mean((y - m) ** 2, axis=-1, keepdims=True)
            outl.append((y - m) * lax.rsqrt(var + 1e-5))
        else:
            outl.append(None)
    if layer == 0:
        return outl
    return outl[-1] @ Wmlp + bmlp


BV = 400
GRID = V // BV


def _make_tc_layer(layer):
    def body(x0r, x1r, x2r, x3r, aggr, degr, Wlr, Wrr, bcr, Wresr, bresr,
             deltar, lognATr, wmixr, bmixr, Wmlpr, bmlpr, *outs):
        xs = [x0r[...], x1r[...], x2r[...], x3r[...]]
        AT = -jnp.exp(lognATr[...])
        res = _dense_body(layer, xs, aggr[...], degr[...], Wlr[...], Wrr[...],
                          bcr[...], Wresr[...], bresr[...], deltar[...], AT,
                          wmixr[...], bmixr[...], Wmlpr[...], bmlpr[...])
        if layer == 0:
            for o_ref, o in zip(outs, res):
                o_ref[...] = o
        else:
            outs[0][...] = res

    bspec = pl.BlockSpec((BV, D), lambda i: (i, 0))
    full = lambda shape: pl.BlockSpec(shape, lambda i: tuple(0 for _ in shape))
    in_specs = [
        bspec, bspec, bspec, bspec,
        pl.BlockSpec((NC, T, BV, D), lambda i: (0, 0, i, 0)),
        pl.BlockSpec((NC, T, BV, D), lambda i: (0, 0, i, 0)),
        full((D, OUT)), full((D, OUT)), full((1, OUT)),
        full((D, D)), full((1, D)),
        full((1, D)), full((N, D)), full((3, D)), full((1, D)),
        full((D, D)), full((1, D)),
    ]
    if layer == 0:
        out_shape = [jax.ShapeDtypeStruct((V, D), jnp.float32)] * T
        out_specs = [bspec] * T
    else:
        out_shape = jax.ShapeDtypeStruct((V, D), jnp.float32)
        out_specs = bspec
    return pl.pallas_call(
        body, grid=(GRID,), in_specs=in_specs, out_specs=out_specs,
        out_shape=out_shape,
        compiler_params=pltpu.CompilerParams(
            dimension_semantics=("arbitrary",)),
    )


_tc_layer0 = _make_tc_layer(0)
_tc_layer1 = _make_tc_layer(1)


def kernel(x0, x1, x2, x3, edge_index0, edge_index1, edge_index2, edge_index3,
           log_nA, delta, Wl0, Wr0, bc0, Wl1, Wr1, bc1, Wres0, bres0,
           Wres1, bres1, wmix, bmix, Wmlp, bmlp):
    eis = (edge_index0, edge_index1, edge_index2, edge_index3)
    srcs = [ei[0] for ei in eis]
    dsts = [ei[1] for ei in eis]
    zrows = jnp.zeros((ZCH, D), jnp.float32)
    ones128 = jnp.ones((KB, D), jnp.float32)

    aggp0, degp = _make_sc_segsum(True)(x0, x1, x2, x3, *srcs, *dsts,
                                        zrows, ones128)

    r2 = lambda b: b.reshape(1, -1)
    ys = _tc_layer0(x0, x1, x2, x3, aggp0, degp,
                    Wl0, Wr0, r2(bc0), Wres0, r2(bres0),
                    r2(delta[0]), log_nA[0].T, wmix, r2(bmix), Wmlp, r2(bmlp))
    y0, y1, y2, y3 = ys

    aggp1 = _make_sc_segsum(False)(y0, y1, y2, y3, *srcs, *dsts,
                                   zrows, ones128)
    if isinstance(aggp1, (list, tuple)):
        aggp1 = aggp1[0]

    out = _tc_layer1(y0, y1, y2, y3, aggp1, degp,
                     Wl1, Wr1, r2(bc1), Wres1, r2(bres1),
                     r2(delta[1]), log_nA[1].T, wmix, r2(bmix), Wmlp, r2(bmlp))
    return out
